# Initial kernel scaffold; baseline (speedup 1.0000x reference)
import jax
import jax.numpy as jnp
from jax.experimental import pallas as pl


def kernel(logits, labels):
    raise NotImplementedError("write your pallas kernel here")


# speedup vs baseline: 1.0020x; 1.0020x over previous
import functools

import jax
import jax.numpy as jnp
from jax.experimental import pallas as pl

S = 64.0
M = 0.4

_BLOCK_COLS = 2048


def _cosface_block(labels_ref, logits_ref, out_ref):
    pid = pl.program_id(0)
    block = logits_ref[...]
    rows, cols = block.shape
    col_ids = jax.lax.broadcasted_iota(jnp.int32, (rows, cols), 1) + pid * cols
    mask = col_ids == labels_ref[...]
    out_ref[...] = block * S - jnp.where(mask, M * S, 0.0)


@jax.jit
def kernel(logits, labels):
    B, V = logits.shape
    labels2d = labels.astype(jnp.int32).reshape(B, 1)
    grid = (pl.cdiv(V, _BLOCK_COLS),)
    return pl.pallas_call(
        _cosface_block,
        grid=grid,
        in_specs=[
            pl.BlockSpec((B, 1), lambda i: (0, 0)),
            pl.BlockSpec((B, _BLOCK_COLS), lambda i: (0, i)),
        ],
        out_specs=pl.BlockSpec((B, _BLOCK_COLS), lambda i: (0, i)),
        out_shape=jax.ShapeDtypeStruct((B, V), logits.dtype),
    )(labels2d, logits)
